# Initial kernel scaffold; baseline (speedup 1.0000x reference)
import jax
import jax.numpy as jnp
from jax.experimental import pallas as pl


def kernel(at_no, shifts):
    raise NotImplementedError("write your pallas kernel here")


# speedup vs baseline: 138.8051x; 138.8051x over previous
import functools

import jax
import jax.numpy as jnp
from jax import lax
from jax.experimental import pallas as pl
from jax.experimental.pallas import tpu as pltpu, tpu_sc as plsc

_L = 16
_NW = 32
_TABLE_PAD = 128


@functools.cache
def _build(chunk: int):
    npad = chunk * _NW
    mesh = plsc.VectorSubcoreMesh(core_axis_name="c", subcore_axis_name="s")

    @functools.partial(
        pl.kernel,
        mesh=mesh,
        compiler_params=pltpu.CompilerParams(needs_layout_passes=False),
        out_type=jax.ShapeDtypeStruct((npad,), jnp.float32),
        scratch_types=[
            pltpu.VMEM((chunk,), jnp.int32),
            pltpu.VMEM((chunk,), jnp.float32),
            pltpu.VMEM((_TABLE_PAD,), jnp.float32),
        ],
    )
    def k(idx_hbm, table_hbm, out_hbm, idx_v, out_v, table_v):
        wid = lax.axis_index("s") * 2 + lax.axis_index("c")
        base = wid * chunk
        pltpu.sync_copy(table_hbm, table_v)
        pltpu.sync_copy(idx_hbm.at[pl.ds(base, chunk)], idx_v)

        def body(j, carry):
            off = pl.multiple_of(j * _L, _L)
            iv = idx_v[pl.ds(off, _L)]
            vals = plsc.load_gather(table_v, [iv])
            out_v[pl.ds(off, _L)] = vals
            return carry

        lax.fori_loop(0, chunk // _L, body, 0)
        pltpu.sync_copy(out_v, out_hbm.at[pl.ds(base, chunk)])

    return k


def kernel(at_no, shifts):
    n = at_no.shape[0]
    grain = _NW * _L
    npad = ((n + grain - 1) // grain) * grain
    chunk = npad // _NW
    idx = jnp.pad(at_no.astype(jnp.int32), (0, npad - n))
    table = jnp.pad(shifts.reshape(-1).astype(jnp.float32),
                    (0, _TABLE_PAD - shifts.shape[0]))
    out = _build(chunk)(idx, table)
    return out[:n].reshape(n, 1)


# speedup vs baseline: 167.4151x; 1.2061x over previous
import functools

import jax
import jax.numpy as jnp
from jax import lax
from jax.experimental import pallas as pl
from jax.experimental.pallas import tpu as pltpu, tpu_sc as plsc

_L = 16
_NW = 32
_TABLE_PAD = 128


@functools.cache
def _build(chunk: int):
    npad = chunk * _NW
    mesh = plsc.VectorSubcoreMesh(core_axis_name="c", subcore_axis_name="s")

    @functools.partial(
        pl.kernel,
        mesh=mesh,
        compiler_params=pltpu.CompilerParams(needs_layout_passes=False),
        out_type=jax.ShapeDtypeStruct((npad,), jnp.float32),
        scratch_types=[
            pltpu.VMEM((chunk,), jnp.int32),
            pltpu.VMEM((chunk,), jnp.float32),
            pltpu.VMEM((_TABLE_PAD,), jnp.float32),
        ],
    )
    def k(idx_hbm, table_hbm, out_hbm, idx_v, out_v, table_v):
        wid = lax.axis_index("s") * 2 + lax.axis_index("c")
        base = wid * chunk
        pltpu.sync_copy(table_hbm, table_v)
        pltpu.sync_copy(idx_hbm.at[pl.ds(base, chunk)], idx_v)

        @plsc.parallel_loop(0, chunk, step=_L, unroll=8)
        def body(off):
            off = pl.multiple_of(off, _L)
            iv = idx_v[pl.ds(off, _L)]
            vals = plsc.load_gather(table_v, [iv])
            out_v[pl.ds(off, _L)] = vals
        pltpu.sync_copy(out_v, out_hbm.at[pl.ds(base, chunk)])

    return k


def kernel(at_no, shifts):
    n = at_no.shape[0]
    grain = _NW * _L
    npad = ((n + grain - 1) // grain) * grain
    chunk = npad // _NW
    idx = jnp.pad(at_no.astype(jnp.int32), (0, npad - n))
    table = jnp.pad(shifts.reshape(-1).astype(jnp.float32),
                    (0, _TABLE_PAD - shifts.shape[0]))
    out = _build(chunk)(idx, table)
    return out[:n].reshape(n, 1)
